# Initial kernel scaffold; baseline (speedup 1.0000x reference)
import jax
import jax.numpy as jnp
from jax.experimental import pallas as pl


def kernel(x, table):
    raise NotImplementedError("write your pallas kernel here")


# speedup vs baseline: 3.0772x; 3.0772x over previous
import functools

import jax
import jax.numpy as jnp
from jax import lax
from jax.experimental import pallas as pl
from jax.experimental.pallas import tpu as pltpu
from jax.experimental.pallas import tpu_sc as plsc

D_MODEL = 128
NUM_CORES = 2
NUM_SUBCORES = 16
NUM_WORKERS = NUM_CORES * NUM_SUBCORES
CHUNK = 128


@functools.lru_cache(maxsize=None)
def _build(nchunk: int, d: int):
    @functools.partial(
        pl.kernel,
        mesh=plsc.VectorSubcoreMesh(core_axis_name="c", subcore_axis_name="s"),
        out_type=jax.ShapeDtypeStruct((NUM_WORKERS * nchunk * CHUNK, d), jnp.float32),
        scratch_types=[
            pltpu.VMEM((nchunk, CHUNK), jnp.int32),
            pltpu.VMEM((CHUNK, d), jnp.float32),
            pltpu.SemaphoreType.DMA,
        ],
    )
    def emb_kernel(x_hbm, table_hbm, out_hbm, idx_v, rows, gsem):
        wid = lax.axis_index("s") * NUM_CORES + lax.axis_index("c")
        base = wid * nchunk * CHUNK
        pltpu.sync_copy(x_hbm.at[wid], idx_v)

        @pl.loop(0, nchunk)
        def _(j):
            pltpu.async_copy(table_hbm.at[idx_v.at[j]], rows, gsem).wait()
            pltpu.sync_copy(rows, out_hbm.at[pl.ds(base + j * CHUNK, CHUNK)])

    return emb_kernel


def kernel(x, table):
    b, t = x.shape
    total = b * t
    nchunk = total // (NUM_WORKERS * CHUNK)
    assert nchunk * NUM_WORKERS * CHUNK == total
    d = table.shape[1]
    x_r = x.reshape(NUM_WORKERS, nchunk, CHUNK).astype(jnp.int32)
    out = _build(nchunk, d)(x_r, table)
    return out.reshape(b, t, d)


# speedup vs baseline: 3.4507x; 1.1214x over previous
import functools

import jax
import jax.numpy as jnp
from jax import lax
from jax.experimental import pallas as pl
from jax.experimental.pallas import tpu as pltpu
from jax.experimental.pallas import tpu_sc as plsc

D_MODEL = 128
NUM_CORES = 2
NUM_SUBCORES = 16
NUM_WORKERS = NUM_CORES * NUM_SUBCORES
CHUNK = 128
CPG = 2


@functools.lru_cache(maxsize=None)
def _build(nchunk: int, d: int):
    ngroup = nchunk // CPG
    assert ngroup * CPG == nchunk and ngroup >= 3 and ngroup % 2 == 1

    @functools.partial(
        pl.kernel,
        mesh=plsc.VectorSubcoreMesh(core_axis_name="c", subcore_axis_name="s"),
        out_type=jax.ShapeDtypeStruct((NUM_WORKERS * nchunk * CHUNK, d), jnp.float32),
        scratch_types=[
            pltpu.VMEM((nchunk, CHUNK), jnp.int32),
        ]
        + [pltpu.VMEM((CHUNK, d), jnp.float32)] * (2 * CPG)
        + [pltpu.SemaphoreType.DMA] * (4 * CPG),
    )
    def emb_kernel(x_hbm, table_hbm, out_hbm, idx_v, *scratch):
        bufs = scratch[: 2 * CPG]
        gsems = scratch[2 * CPG : 4 * CPG]
        osems = scratch[4 * CPG :]
        pair0 = tuple(range(CPG))
        pair1 = tuple(range(CPG, 2 * CPG))

        wid = lax.axis_index("s") * NUM_CORES + lax.axis_index("c")
        base = wid * nchunk * CHUNK
        pltpu.sync_copy(x_hbm.at[wid], idx_v)

        def fire_gather(chunk, b):
            pltpu.async_copy(table_hbm.at[idx_v.at[chunk]], bufs[b], gsems[b])

        def wait_gather(b):
            pltpu.make_async_copy(
                table_hbm.at[idx_v.at[0]], bufs[b], gsems[b]
            ).wait()

        def fire_out(chunk, b):
            pltpu.async_copy(
                bufs[b], out_hbm.at[pl.ds(base + chunk * CHUNK, CHUNK)], osems[b]
            )

        def wait_out(b):
            pltpu.make_async_copy(
                bufs[b], out_hbm.at[pl.ds(base, CHUNK)], osems[b]
            ).wait()

        def phase(g, cur, nxt, first=False, last=False):
            for b in cur:
                wait_gather(b)
            if not first:
                for b in nxt:
                    wait_out(b)
            if not last:
                for i, b in enumerate(nxt):
                    fire_gather((g + 1) * CPG + i, b)
            for i, b in enumerate(cur):
                fire_out(g * CPG + i, b)

        for i, b in enumerate(pair0):
            fire_gather(i, b)
        phase(0, pair0, pair1, first=True)

        @pl.loop(1, ngroup - 2, step=2)
        def _(g):
            phase(g, pair1, pair0)
            phase(g + 1, pair0, pair1)

        phase(ngroup - 2, pair1, pair0)
        phase(ngroup - 1, pair0, pair1, last=True)
        for b in pair0:
            wait_out(b)

    return emb_kernel


def kernel(x, table):
    b, t = x.shape
    total = b * t
    nchunk = total // (NUM_WORKERS * CHUNK)
    assert nchunk * NUM_WORKERS * CHUNK == total
    d = table.shape[1]
    x_r = x.reshape(NUM_WORKERS, nchunk, CHUNK).astype(jnp.int32)
    out = _build(nchunk, d)(x_r, table)
    return out.reshape(b, t, d)


# speedup vs baseline: 6.0863x; 1.7638x over previous
import functools

import jax
import jax.numpy as jnp
from jax import lax
from jax.experimental import pallas as pl
from jax.experimental.pallas import tpu as pltpu
from jax.experimental.pallas import tpu_sc as plsc

NUM_CORES = 2
NUM_SUBCORES = 16
NUM_WORKERS = NUM_CORES * NUM_SUBCORES
CPG = 4


@functools.lru_cache(maxsize=None)
def _build(batch: int, seq: int, d: int):
    rows_per_w = batch // NUM_WORKERS
    ngroup = rows_per_w // CPG
    assert ngroup * CPG == rows_per_w and ngroup >= 4 and ngroup % 2 == 0

    @functools.partial(
        pl.kernel,
        mesh=plsc.VectorSubcoreMesh(core_axis_name="c", subcore_axis_name="s"),
        out_type=jax.ShapeDtypeStruct((batch, seq, d), jnp.float32),
        scratch_types=[
            pltpu.VMEM((rows_per_w, seq), jnp.int32),
        ]
        + [pltpu.VMEM((seq, d), jnp.float32)] * (2 * CPG)
        + [pltpu.SemaphoreType.DMA] * (4 * CPG),
    )
    def emb_kernel(x_hbm, table_hbm, out_hbm, idx_v, *scratch):
        bufs = scratch[: 2 * CPG]
        gsems = scratch[2 * CPG : 4 * CPG]
        osems = scratch[4 * CPG :]
        pair0 = tuple(range(CPG))
        pair1 = tuple(range(CPG, 2 * CPG))

        wid = lax.axis_index("s") * NUM_CORES + lax.axis_index("c")
        base = wid * rows_per_w
        pltpu.sync_copy(x_hbm.at[pl.ds(base, rows_per_w)], idx_v)

        def fire_gather(row, b):
            pltpu.async_copy(table_hbm.at[idx_v.at[row]], bufs[b], gsems[b])

        def wait_gather(b):
            pltpu.make_async_copy(
                table_hbm.at[idx_v.at[0]], bufs[b], gsems[b]
            ).wait()

        def fire_out(row, b):
            pltpu.async_copy(bufs[b], out_hbm.at[base + row], osems[b])

        def wait_out(b):
            pltpu.make_async_copy(bufs[b], out_hbm.at[base], osems[b]).wait()

        def phase(g, cur, nxt, first=False, last=False):
            for b in cur:
                wait_gather(b)
            if not first:
                for b in nxt:
                    wait_out(b)
            if not last:
                for i, b in enumerate(nxt):
                    fire_gather((g + 1) * CPG + i, b)
            for i, b in enumerate(cur):
                fire_out(g * CPG + i, b)

        for i, b in enumerate(pair0):
            fire_gather(i, b)
        phase(0, pair0, pair1, first=True)

        @pl.loop(1, ngroup - 1, step=2)
        def _(g):
            phase(g, pair1, pair0)
            phase(g + 1, pair0, pair1)

        phase(ngroup - 1, pair1, pair0, last=True)
        for b in pair1:
            wait_out(b)

    return emb_kernel


def kernel(x, table):
    batch, seq = x.shape
    d = table.shape[1]
    assert batch % NUM_WORKERS == 0
    return _build(batch, seq, d)(x.astype(jnp.int32), table)
